# baseline (speedup 1.0000x reference)
import functools

import jax
import jax.numpy as jnp
from jax import lax
from jax.experimental import pallas as pl
from jax.experimental.pallas import tpu as pltpu
from jax.experimental.pallas import tpu_sc as plsc

N = 50000
D = 128
E = 300000
P = 8192
NC, NS = 2, 16
C = 768
K = 25
ET = K * C
EP = NS * ET
NACC = 50176
SLC = NACC // NS
NT = 51200
G = 8
GW = D // G
NT8 = NT // G
UBLK = 1024
TBLK = 320

_sc_mesh = None


def _mesh():
    global _sc_mesh
    if _sc_mesh is None:
        _sc_mesh = plsc.VectorSubcoreMesh(core_axis_name="c", subcore_axis_name="s")
    return _sc_mesh


def _ln_gelu(y, g, b):
    m = jnp.mean(y, axis=-1, keepdims=True)
    v = jnp.mean((y - m) ** 2, axis=-1, keepdims=True)
    y = (y - m) * lax.rsqrt(v + 1e-5) * g + b
    return y * 0.5 * (1.0 + lax.erf(y * 0.7071067811865476))


def _proj(x, W, b, g, beta):
    def body(x_ref, w_ref, b_ref, g_ref, beta_ref, o_ref):
        y = jnp.dot(x_ref[...], w_ref[...], preferred_element_type=jnp.float32)
        y = y + b_ref[...]
        o_ref[...] = _ln_gelu(y, g_ref[...], beta_ref[...])

    return pl.pallas_call(
        body,
        grid=(NACC // UBLK,),
        in_specs=[pl.BlockSpec((UBLK, D), lambda i: (i, 0)),
                  pl.BlockSpec((D, D), lambda i: (0, 0)),
                  pl.BlockSpec((1, D), lambda i: (0, 0)),
                  pl.BlockSpec((1, D), lambda i: (0, 0)),
                  pl.BlockSpec((1, D), lambda i: (0, 0))],
        out_specs=pl.BlockSpec((UBLK, D), lambda i: (i, 0)),
        out_shape=jax.ShapeDtypeStruct((NACC, D), jnp.float32),
    )(x, W.astype(jnp.float32), b.reshape(1, D), g.reshape(1, D),
      beta.reshape(1, D))


def _transform(h, W):
    h8 = jnp.pad(h, ((0, NT - h.shape[0]), (0, 0))).reshape(NT8, G * D)
    eye8 = jnp.eye(G, dtype=jnp.float32)
    wg = W.astype(jnp.float32).reshape(D, G, GW).transpose(1, 0, 2)
    wd = jnp.einsum('ab,gkc->gakbc', eye8, wg).reshape(G, G * D, D)

    def body(h8_ref, wd_ref, o_ref):
        o_ref[0] = jnp.dot(h8_ref[...], wd_ref[0],
                           preferred_element_type=jnp.float32)

    t4p = pl.pallas_call(
        body,
        grid=(NT8 // TBLK, G),
        in_specs=[pl.BlockSpec((TBLK, G * D), lambda i, g: (i, 0)),
                  pl.BlockSpec((1, G * D, D), lambda i, g: (g, 0, 0))],
        out_specs=pl.BlockSpec((1, TBLK, D), lambda i, g: (g, i, 0)),
        out_shape=jax.ShapeDtypeStruct((G, NT8, D), jnp.float32),
    )(h8, wd)
    return t4p.reshape(G, NT, GW)


def _counts(dcd, ddc, z1):
    @functools.partial(
        pl.kernel,
        out_type=(jax.ShapeDtypeStruct((NACC,), jnp.float32),
                  jax.ShapeDtypeStruct((NACC,), jnp.float32)),
        mesh=_mesh(),
        compiler_params=pltpu.CompilerParams(use_tc_tiling_on_sc=False),
        scratch_types=[
            pltpu.VMEM((C,), jnp.int32),
            pltpu.VMEM((C,), jnp.float32),
            pltpu.VMEM_SHARED((NACC,), jnp.float32),
            pltpu.SemaphoreType.DMA,
        ],
    )
    def kern(dcd_hbm, ddc_hbm, z1_hbm, outd_hbm, outc_hbm,
             didx_v, ones_v, acc, sem):
        cid = lax.axis_index("c")
        sid = lax.axis_index("s")
        for i in range(C // 16):
            ones_v[pl.ds(i * 16, 16)] = jnp.ones((16,), jnp.float32)
        for cc in range(NC):
            @pl.when(cid == cc)
            def _():
                eref = dcd_hbm if cc == 0 else ddc_hbm
                oref = outd_hbm if cc == 0 else outc_hbm
                pltpu.sync_copy(z1_hbm, acc.at[pl.ds(sid * SLC, SLC)])
                plsc.subcore_barrier()
                for k in range(K):
                    pltpu.sync_copy(eref.at[sid * K + k], didx_v)
                    pltpu.sync_copy(ones_v, acc.at[didx_v], add=True)
                plsc.subcore_barrier()
                pltpu.sync_copy(acc.at[pl.ds(sid * SLC, SLC)],
                                oref.at[pl.ds(sid * SLC, SLC)])

    return kern(dcd, ddc, z1)


def _segsum(t4, sidx2, didx2, z):
    @functools.partial(
        pl.kernel,
        out_type=jax.ShapeDtypeStruct((NACC, D), jnp.float32),
        mesh=_mesh(),
        compiler_params=pltpu.CompilerParams(use_tc_tiling_on_sc=False),
        scratch_types=[
            pltpu.VMEM((K, C), jnp.int32),
            pltpu.VMEM((K, C), jnp.int32),
            pltpu.VMEM((2, C, GW), jnp.float32),
            pltpu.VMEM_SHARED((NACC, GW), jnp.float32),
            pltpu.SemaphoreType.DMA,
            pltpu.SemaphoreType.DMA,
            pltpu.SemaphoreType.DMA,
            pltpu.SemaphoreType.DMA,
        ],
    )
    def kern(t4_hbm, sidx_hbm, didx_hbm, z_hbm, out_hbm,
             idxs_v, didxs_v, rows_v, acc, g0, g1, s0, s1):
        cid = lax.axis_index("c")
        sid = lax.axis_index("s")
        gsem = (g0, g1)
        ssem = (s0, s1)
        pltpu.sync_copy(sidx_hbm.at[pl.ds(sid * K, K)], idxs_v)
        pltpu.sync_copy(didx_hbm.at[pl.ds(sid * K, K)], didxs_v)
        for cc in range(NC):
            @pl.when(cid == cc)
            def _():
                for gg in range(G // NC):
                    g = (G // NC) * cc + gg
                    pltpu.sync_copy(z_hbm, acc.at[pl.ds(sid * SLC, SLC)])
                    plsc.subcore_barrier()
                    gd = [None] * K
                    sd = [None] * K
                    for k in range(K):
                        b = k & 1
                        if k >= 2:
                            sd[k - 2].wait()
                        gd[k] = pltpu.async_copy(
                            t4_hbm.at[g].at[idxs_v.at[k]], rows_v.at[b],
                            gsem[b])
                        if k >= 1:
                            gd[k - 1].wait()
                            sd[k - 1] = pltpu.async_copy(
                                rows_v.at[(k - 1) & 1],
                                acc.at[didxs_v.at[k - 1]],
                                ssem[(k - 1) & 1], add=True)
                    gd[K - 1].wait()
                    sd[K - 1] = pltpu.async_copy(
                        rows_v.at[(K - 1) & 1], acc.at[didxs_v.at[K - 1]],
                        ssem[(K - 1) & 1], add=True)
                    sd[K - 2].wait()
                    sd[K - 1].wait()
                    plsc.subcore_barrier()
                    pltpu.sync_copy(
                        acc.at[pl.ds(sid * SLC, SLC)],
                        out_hbm.at[pl.ds(sid * SLC, SLC), pl.ds(g * GW, GW)])
                    plsc.subcore_barrier()

    return kern(t4, sidx2, didx2, z)


def _update(S, cnt, h, Wr, bl, g, b):
    def body(s_ref, c_ref, h_ref, w_ref, bl_ref, g_ref, b_ref, o_ref):
        rcp = 1.0 / jnp.maximum(c_ref[...], 1.0)
        hb = h_ref[...]
        u = s_ref[...] * rcp[:, None] + bl_ref[...]
        u = u + jnp.dot(hb, w_ref[...], preferred_element_type=jnp.float32) + hb
        o_ref[...] = _ln_gelu(u, g_ref[...], b_ref[...])

    return pl.pallas_call(
        body,
        grid=(NACC // UBLK,),
        in_specs=[pl.BlockSpec((UBLK, D), lambda i: (i, 0)),
                  pl.BlockSpec((UBLK,), lambda i: (i,)),
                  pl.BlockSpec((UBLK, D), lambda i: (i, 0)),
                  pl.BlockSpec((D, D), lambda i: (0, 0)),
                  pl.BlockSpec((1, D), lambda i: (0, 0)),
                  pl.BlockSpec((1, D), lambda i: (0, 0)),
                  pl.BlockSpec((1, D), lambda i: (0, 0))],
        out_specs=pl.BlockSpec((UBLK, D), lambda i: (i, 0)),
        out_shape=jax.ShapeDtypeStruct((NACC, D), jnp.float32),
    )(S, cnt, h, Wr.astype(jnp.float32), bl.reshape(1, D), g.reshape(1, D),
      b.reshape(1, D))


def _decode_gather(h_ch, h_di, pos0, pos1, neg0, neg1):
    GC = 256
    per_w = P // 8

    @functools.partial(
        pl.kernel,
        out_type=jax.ShapeDtypeStruct((4, P, D), jnp.float32),
        mesh=_mesh(),
        compiler_params=pltpu.CompilerParams(use_tc_tiling_on_sc=False),
        scratch_types=[
            pltpu.VMEM((GC,), jnp.int32),
            pltpu.VMEM((GC, D), jnp.float32),
            pltpu.SemaphoreType.DMA,
        ],
    )
    def kern(hch_hbm, hdi_hbm, p0_hbm, p1_hbm, n0_hbm, n1_hbm, out_hbm,
             idx_v, rows_v, sem):
        cid = lax.axis_index("c")
        sid = lax.axis_index("s")
        wid = sid * NC + cid
        a = wid % 4
        j = wid // 4
        tables = (hch_hbm, hdi_hbm, hch_hbm, hdi_hbm)
        ilists = (p0_hbm, p1_hbm, n0_hbm, n1_hbm)
        for aa in range(4):
            @pl.when(a == aa)
            def _():
                for ch in range(per_w // GC):
                    base = j * per_w + ch * GC
                    pltpu.sync_copy(ilists[aa].at[pl.ds(base, GC)], idx_v)
                    pltpu.async_copy(tables[aa].at[idx_v], rows_v, sem).wait()
                    pltpu.sync_copy(rows_v, out_hbm.at[aa].at[pl.ds(base, GC)])

    return kern(h_ch, h_di, pos0, pos1, neg0, neg1)


def _rowdot(g0, g1, g2, g3, W):
    RB = 1024

    def body(a_ref, b_ref, c_ref, d_ref, w_ref, po_ref, no_ref):
        w = w_ref[...]
        s = jnp.dot(a_ref[...], w, preferred_element_type=jnp.float32)
        po_ref[...] = jnp.sum(s * b_ref[...], axis=1)
        t = jnp.dot(c_ref[...], w, preferred_element_type=jnp.float32)
        no_ref[...] = jnp.sum(t * d_ref[...], axis=1)

    return pl.pallas_call(
        body,
        grid=(P // RB,),
        in_specs=[pl.BlockSpec((RB, D), lambda i: (i, 0)),
                  pl.BlockSpec((RB, D), lambda i: (i, 0)),
                  pl.BlockSpec((RB, D), lambda i: (i, 0)),
                  pl.BlockSpec((RB, D), lambda i: (i, 0)),
                  pl.BlockSpec((D, D), lambda i: (0, 0))],
        out_specs=[pl.BlockSpec((RB,), lambda i: (i,)),
                   pl.BlockSpec((RB,), lambda i: (i,))],
        out_shape=[jax.ShapeDtypeStruct((P,), jnp.float32),
                   jax.ShapeDtypeStruct((P,), jnp.float32)],
    )(g0, g1, g2, g3, W.astype(jnp.float32))


def _pad_edges(ei):
    pad = EP - E
    ar = lax.iota(jnp.int32, pad)
    src = jnp.concatenate([ei[0].astype(jnp.int32), (ar * 7) % N])
    dst = jnp.concatenate([ei[1].astype(jnp.int32), N + ar % (NACC - N)])
    return src.reshape(EP // C, C), dst.reshape(EP // C, C)


def kernel(x_chem, proj_W, proj_b, proj_g, proj_beta, emb_dis,
           Wl_cd0, bl_cd0, Wr_cd0, Wl_dc0, bl_dc0, Wr_dc0,
           g_ch0, b_ch0, g_di0, b_di0,
           Wl_cd1, bl_cd1, Wr_cd1, Wl_dc1, bl_dc1, Wr_dc1,
           g_ch1, b_ch1, g_di1, b_di1,
           W_cd, edge_index_cd, edge_index_dc, pos_edge_idx, neg_edge_idx):
    src_cd, dst_cd = _pad_edges(edge_index_cd)
    src_dc, dst_dc = _pad_edges(edge_index_dc)
    z1 = jnp.zeros((SLC,), jnp.float32)
    z = jnp.zeros((SLC, GW), jnp.float32)

    cnt_di, cnt_ch = _counts(dst_cd, dst_dc, z1)

    x_pad = jnp.pad(x_chem.astype(jnp.float32), ((0, NACC - N), (0, 0)))
    h_ch = _proj(x_pad, proj_W, proj_b, proj_g, proj_beta)
    h_di = jnp.pad(emb_dis.astype(jnp.float32), ((0, NACC - N), (0, 0)))

    layer_params = (
        (Wl_cd0, bl_cd0, Wr_cd0, Wl_dc0, bl_dc0, Wr_dc0,
         g_ch0, b_ch0, g_di0, b_di0),
        (Wl_cd1, bl_cd1, Wr_cd1, Wl_dc1, bl_dc1, Wr_dc1,
         g_ch1, b_ch1, g_di1, b_di1),
    )
    for (Wl_cd, bl_cd, Wr_cd, Wl_dc, bl_dc, Wr_dc,
         g_ch, b_ch, g_di, b_di) in layer_params:
        t_cd = _transform(h_ch, Wl_cd)
        S_cd = _segsum(t_cd, src_cd, dst_cd, z)
        t_dc = _transform(h_di, Wl_dc)
        S_dc = _segsum(t_dc, src_dc, dst_dc, z)
        h_di_new = _update(S_cd, cnt_di, h_di, Wr_cd, bl_cd, g_di, b_di)
        h_ch_new = _update(S_dc, cnt_ch, h_ch, Wr_dc, bl_dc, g_ch, b_ch)
        h_ch, h_di = h_ch_new, h_di_new

    pos0 = pos_edge_idx[0].astype(jnp.int32)
    pos1 = pos_edge_idx[1].astype(jnp.int32)
    neg0 = neg_edge_idx[0].astype(jnp.int32)
    neg1 = neg_edge_idx[1].astype(jnp.int32)
    G = _decode_gather(h_ch, h_di, pos0, pos1, neg0, neg1)
    pos, neg = _rowdot(G[0], G[1], G[2], G[3], W_cd)
    return pos, neg


# speedup vs baseline: 1.1312x; 1.1312x over previous
import functools

import jax
import jax.numpy as jnp
from jax import lax
from jax.experimental import pallas as pl
from jax.experimental.pallas import tpu as pltpu
from jax.experimental.pallas import tpu_sc as plsc

N = 50000
D = 128
E = 300000
P = 8192
NC, NS = 2, 16
C = 384
K = 49
ET = K * C
EP = NS * ET
NACC = 50176
SLC = NACC // NS
NT = 51200
G = 4
GW = D // G
NT8 = NT // G
UBLK = 1024
TBLK = 512

_sc_mesh = None


def _mesh():
    global _sc_mesh
    if _sc_mesh is None:
        _sc_mesh = plsc.VectorSubcoreMesh(core_axis_name="c", subcore_axis_name="s")
    return _sc_mesh


def _ln_gelu(y, g, b):
    m = jnp.mean(y, axis=-1, keepdims=True)
    v = jnp.mean((y - m) ** 2, axis=-1, keepdims=True)
    y = (y - m) * lax.rsqrt(v + 1e-5) * g + b
    return y * 0.5 * (1.0 + lax.erf(y * 0.7071067811865476))


def _proj(x, W, b, g, beta):
    def body(x_ref, w_ref, b_ref, g_ref, beta_ref, o_ref):
        y = jnp.dot(x_ref[...], w_ref[...], preferred_element_type=jnp.float32)
        y = y + b_ref[...]
        o_ref[...] = _ln_gelu(y, g_ref[...], beta_ref[...])

    return pl.pallas_call(
        body,
        grid=(NACC // UBLK,),
        in_specs=[pl.BlockSpec((UBLK, D), lambda i: (i, 0)),
                  pl.BlockSpec((D, D), lambda i: (0, 0)),
                  pl.BlockSpec((1, D), lambda i: (0, 0)),
                  pl.BlockSpec((1, D), lambda i: (0, 0)),
                  pl.BlockSpec((1, D), lambda i: (0, 0))],
        out_specs=pl.BlockSpec((UBLK, D), lambda i: (i, 0)),
        out_shape=jax.ShapeDtypeStruct((NACC, D), jnp.float32),
    )(x, W.astype(jnp.float32), b.reshape(1, D), g.reshape(1, D),
      beta.reshape(1, D))


def _transform(h, W):
    h8 = jnp.pad(h, ((0, NT - h.shape[0]), (0, 0))).reshape(NT8, G * D)
    eye8 = jnp.eye(G, dtype=jnp.float32)
    wg = W.astype(jnp.float32).reshape(D, G, GW).transpose(1, 0, 2)
    wd = jnp.einsum('ab,gkc->gakbc', eye8, wg).reshape(G, G * D, D)

    def body(h8_ref, wd_ref, o_ref):
        o_ref[0] = jnp.dot(h8_ref[...], wd_ref[0],
                           preferred_element_type=jnp.float32)

    t4p = pl.pallas_call(
        body,
        grid=(NT8 // TBLK, G),
        in_specs=[pl.BlockSpec((TBLK, G * D), lambda i, g: (i, 0)),
                  pl.BlockSpec((1, G * D, D), lambda i, g: (g, 0, 0))],
        out_specs=pl.BlockSpec((1, TBLK, D), lambda i, g: (g, i, 0)),
        out_shape=jax.ShapeDtypeStruct((G, NT8, D), jnp.float32),
    )(h8, wd)
    return t4p.reshape(G, NT, GW)


def _counts(dcd, ddc, z1):
    @functools.partial(
        pl.kernel,
        out_type=(jax.ShapeDtypeStruct((NACC,), jnp.float32),
                  jax.ShapeDtypeStruct((NACC,), jnp.float32)),
        mesh=_mesh(),
        compiler_params=pltpu.CompilerParams(use_tc_tiling_on_sc=False),
        scratch_types=[
            pltpu.VMEM((C,), jnp.int32),
            pltpu.VMEM((C,), jnp.float32),
            pltpu.VMEM_SHARED((NACC,), jnp.float32),
            pltpu.SemaphoreType.DMA,
        ],
    )
    def kern(dcd_hbm, ddc_hbm, z1_hbm, outd_hbm, outc_hbm,
             didx_v, ones_v, acc, sem):
        cid = lax.axis_index("c")
        sid = lax.axis_index("s")
        for i in range(C // 16):
            ones_v[pl.ds(i * 16, 16)] = jnp.ones((16,), jnp.float32)
        for cc in range(NC):
            @pl.when(cid == cc)
            def _():
                eref = dcd_hbm if cc == 0 else ddc_hbm
                oref = outd_hbm if cc == 0 else outc_hbm
                pltpu.sync_copy(z1_hbm, acc.at[pl.ds(sid * SLC, SLC)])
                plsc.subcore_barrier()
                for k in range(K):
                    pltpu.sync_copy(eref.at[sid * K + k], didx_v)
                    pltpu.sync_copy(ones_v, acc.at[didx_v], add=True)
                plsc.subcore_barrier()
                pltpu.sync_copy(acc.at[pl.ds(sid * SLC, SLC)],
                                oref.at[pl.ds(sid * SLC, SLC)])

    return kern(dcd, ddc, z1)


def _segsum(t4, sidx2, didx2, z):
    @functools.partial(
        pl.kernel,
        out_type=jax.ShapeDtypeStruct((NACC, D), jnp.float32),
        mesh=_mesh(),
        compiler_params=pltpu.CompilerParams(use_tc_tiling_on_sc=False),
        scratch_types=[
            pltpu.VMEM((2, C), jnp.int32),
            pltpu.VMEM((2, C), jnp.int32),
            pltpu.VMEM((2, C, GW), jnp.float32),
            pltpu.VMEM_SHARED((NACC, GW), jnp.float32),
            pltpu.SemaphoreType.DMA,
            pltpu.SemaphoreType.DMA,
            pltpu.SemaphoreType.DMA,
            pltpu.SemaphoreType.DMA,
        ],
    )
    def kern(t4_hbm, sidx_hbm, didx_hbm, z_hbm, out_hbm,
             idx_v, didx_v, rows_v, acc, g0, g1, s0, s1):
        cid = lax.axis_index("c")
        sid = lax.axis_index("s")
        gsem = (g0, g1)
        ssem = (s0, s1)
        for cc in range(NC):
            @pl.when(cid == cc)
            def _():
                for gg in range(G // NC):
                    g = (G // NC) * cc + gg
                    pltpu.sync_copy(z_hbm, acc.at[pl.ds(sid * SLC, SLC)])
                    plsc.subcore_barrier()
                    gd = [None] * K
                    sd = [None] * K
                    for k in range(K):
                        b = k & 1
                        if k >= 2:
                            sd[k - 2].wait()
                        pltpu.sync_copy(sidx_hbm.at[sid * K + k], idx_v.at[b])
                        pltpu.sync_copy(didx_hbm.at[sid * K + k], didx_v.at[b])
                        gd[k] = pltpu.async_copy(
                            t4_hbm.at[g].at[idx_v.at[b]], rows_v.at[b],
                            gsem[b])
                        if k >= 1:
                            gd[k - 1].wait()
                            sd[k - 1] = pltpu.async_copy(
                                rows_v.at[(k - 1) & 1],
                                acc.at[didx_v.at[(k - 1) & 1]],
                                ssem[(k - 1) & 1], add=True)
                    gd[K - 1].wait()
                    sd[K - 1] = pltpu.async_copy(
                        rows_v.at[(K - 1) & 1], acc.at[didx_v.at[(K - 1) & 1]],
                        ssem[(K - 1) & 1], add=True)
                    sd[K - 2].wait()
                    sd[K - 1].wait()
                    plsc.subcore_barrier()
                    pltpu.sync_copy(
                        acc.at[pl.ds(sid * SLC, SLC)],
                        out_hbm.at[pl.ds(sid * SLC, SLC), pl.ds(g * GW, GW)])
                    plsc.subcore_barrier()

    return kern(t4, sidx2, didx2, z)


def _update(S, cnt, h, Wr, bl, g, b):
    def body(s_ref, c_ref, h_ref, w_ref, bl_ref, g_ref, b_ref, o_ref):
        rcp = 1.0 / jnp.maximum(c_ref[...], 1.0)
        hb = h_ref[...]
        u = s_ref[...] * rcp[:, None] + bl_ref[...]
        u = u + jnp.dot(hb, w_ref[...], preferred_element_type=jnp.float32) + hb
        o_ref[...] = _ln_gelu(u, g_ref[...], b_ref[...])

    return pl.pallas_call(
        body,
        grid=(NACC // UBLK,),
        in_specs=[pl.BlockSpec((UBLK, D), lambda i: (i, 0)),
                  pl.BlockSpec((UBLK,), lambda i: (i,)),
                  pl.BlockSpec((UBLK, D), lambda i: (i, 0)),
                  pl.BlockSpec((D, D), lambda i: (0, 0)),
                  pl.BlockSpec((1, D), lambda i: (0, 0)),
                  pl.BlockSpec((1, D), lambda i: (0, 0)),
                  pl.BlockSpec((1, D), lambda i: (0, 0))],
        out_specs=pl.BlockSpec((UBLK, D), lambda i: (i, 0)),
        out_shape=jax.ShapeDtypeStruct((NACC, D), jnp.float32),
    )(S, cnt, h, Wr.astype(jnp.float32), bl.reshape(1, D), g.reshape(1, D),
      b.reshape(1, D))


def _decode_gather(h_ch, h_di, pos0, pos1, neg0, neg1):
    GC = 256
    per_w = P // 8

    @functools.partial(
        pl.kernel,
        out_type=jax.ShapeDtypeStruct((4, P, D), jnp.float32),
        mesh=_mesh(),
        compiler_params=pltpu.CompilerParams(use_tc_tiling_on_sc=False),
        scratch_types=[
            pltpu.VMEM((GC,), jnp.int32),
            pltpu.VMEM((GC, D), jnp.float32),
            pltpu.SemaphoreType.DMA,
        ],
    )
    def kern(hch_hbm, hdi_hbm, p0_hbm, p1_hbm, n0_hbm, n1_hbm, out_hbm,
             idx_v, rows_v, sem):
        cid = lax.axis_index("c")
        sid = lax.axis_index("s")
        wid = sid * NC + cid
        a = wid % 4
        j = wid // 4
        tables = (hch_hbm, hdi_hbm, hch_hbm, hdi_hbm)
        ilists = (p0_hbm, p1_hbm, n0_hbm, n1_hbm)
        for aa in range(4):
            @pl.when(a == aa)
            def _():
                for ch in range(per_w // GC):
                    base = j * per_w + ch * GC
                    pltpu.sync_copy(ilists[aa].at[pl.ds(base, GC)], idx_v)
                    pltpu.async_copy(tables[aa].at[idx_v], rows_v, sem).wait()
                    pltpu.sync_copy(rows_v, out_hbm.at[aa].at[pl.ds(base, GC)])

    return kern(h_ch, h_di, pos0, pos1, neg0, neg1)


def _rowdot(g0, g1, g2, g3, W):
    RB = 1024

    def body(a_ref, b_ref, c_ref, d_ref, w_ref, po_ref, no_ref):
        w = w_ref[...]
        s = jnp.dot(a_ref[...], w, preferred_element_type=jnp.float32)
        po_ref[...] = jnp.sum(s * b_ref[...], axis=1)
        t = jnp.dot(c_ref[...], w, preferred_element_type=jnp.float32)
        no_ref[...] = jnp.sum(t * d_ref[...], axis=1)

    return pl.pallas_call(
        body,
        grid=(P // RB,),
        in_specs=[pl.BlockSpec((RB, D), lambda i: (i, 0)),
                  pl.BlockSpec((RB, D), lambda i: (i, 0)),
                  pl.BlockSpec((RB, D), lambda i: (i, 0)),
                  pl.BlockSpec((RB, D), lambda i: (i, 0)),
                  pl.BlockSpec((D, D), lambda i: (0, 0))],
        out_specs=[pl.BlockSpec((RB,), lambda i: (i,)),
                   pl.BlockSpec((RB,), lambda i: (i,))],
        out_shape=[jax.ShapeDtypeStruct((P,), jnp.float32),
                   jax.ShapeDtypeStruct((P,), jnp.float32)],
    )(g0, g1, g2, g3, W.astype(jnp.float32))


def _pad_edges(ei):
    pad = EP - E
    ar = lax.iota(jnp.int32, pad)
    src = jnp.concatenate([ei[0].astype(jnp.int32), (ar * 7) % N])
    dst = jnp.concatenate([ei[1].astype(jnp.int32), N + ar % (NACC - N)])
    return src.reshape(EP // C, C), dst.reshape(EP // C, C)


def kernel(x_chem, proj_W, proj_b, proj_g, proj_beta, emb_dis,
           Wl_cd0, bl_cd0, Wr_cd0, Wl_dc0, bl_dc0, Wr_dc0,
           g_ch0, b_ch0, g_di0, b_di0,
           Wl_cd1, bl_cd1, Wr_cd1, Wl_dc1, bl_dc1, Wr_dc1,
           g_ch1, b_ch1, g_di1, b_di1,
           W_cd, edge_index_cd, edge_index_dc, pos_edge_idx, neg_edge_idx):
    src_cd, dst_cd = _pad_edges(edge_index_cd)
    src_dc, dst_dc = _pad_edges(edge_index_dc)
    z1 = jnp.zeros((SLC,), jnp.float32)
    z = jnp.zeros((SLC, GW), jnp.float32)

    cnt_di, cnt_ch = _counts(dst_cd, dst_dc, z1)

    x_pad = jnp.pad(x_chem.astype(jnp.float32), ((0, NACC - N), (0, 0)))
    h_ch = _proj(x_pad, proj_W, proj_b, proj_g, proj_beta)
    h_di = jnp.pad(emb_dis.astype(jnp.float32), ((0, NACC - N), (0, 0)))

    layer_params = (
        (Wl_cd0, bl_cd0, Wr_cd0, Wl_dc0, bl_dc0, Wr_dc0,
         g_ch0, b_ch0, g_di0, b_di0),
        (Wl_cd1, bl_cd1, Wr_cd1, Wl_dc1, bl_dc1, Wr_dc1,
         g_ch1, b_ch1, g_di1, b_di1),
    )
    for (Wl_cd, bl_cd, Wr_cd, Wl_dc, bl_dc, Wr_dc,
         g_ch, b_ch, g_di, b_di) in layer_params:
        t_cd = _transform(h_ch, Wl_cd)
        S_cd = _segsum(t_cd, src_cd, dst_cd, z)
        t_dc = _transform(h_di, Wl_dc)
        S_dc = _segsum(t_dc, src_dc, dst_dc, z)
        h_di_new = _update(S_cd, cnt_di, h_di, Wr_cd, bl_cd, g_di, b_di)
        h_ch_new = _update(S_dc, cnt_ch, h_ch, Wr_dc, bl_dc, g_ch, b_ch)
        h_ch, h_di = h_ch_new, h_di_new

    pos0 = pos_edge_idx[0].astype(jnp.int32)
    pos1 = pos_edge_idx[1].astype(jnp.int32)
    neg0 = neg_edge_idx[0].astype(jnp.int32)
    neg1 = neg_edge_idx[1].astype(jnp.int32)
    G = _decode_gather(h_ch, h_di, pos0, pos1, neg0, neg1)
    pos, neg = _rowdot(G[0], G[1], G[2], G[3], W_cd)
    return pos, neg
